# baseline (speedup 1.0000x reference)
import jax
import jax.numpy as jnp
from jax.experimental import pallas as pl
from jax.experimental.pallas import tpu as pltpu

_CHUNK = 8


def _dma_kernel(x_ref, w_ref, b_ref, o_ref, y_ref, sem):
    y = jax.lax.dot_general(
        x_ref[...], w_ref[...], (((1,), (1,)), ((), ())),
        preferred_element_type=jnp.float32)
    y_ref[...] = jnp.maximum(y + b_ref[...], 0.0)
    n = x_ref.shape[0]

    def issue(i):
        return pltpu.make_async_copy(y_ref, o_ref.at[i], sem)

    def body(g, _):
        base = g * _CHUNK
        for j in range(_CHUNK):
            issue(base + j).start()
        for j in range(_CHUNK):
            issue(base + j).wait()
        return 0

    jax.lax.fori_loop(0, n // _CHUNK, body, 0)


def kernel(X, sample_domain, W, b):
    n, d = X.shape
    out = pl.pallas_call(
        _dma_kernel,
        in_specs=[
            pl.BlockSpec(memory_space=pltpu.VMEM),
            pl.BlockSpec(memory_space=pltpu.VMEM),
            pl.BlockSpec(memory_space=pltpu.VMEM),
        ],
        out_specs=pl.BlockSpec(memory_space=pl.ANY),
        out_shape=jax.ShapeDtypeStruct((n, n, d), jnp.float32),
        scratch_shapes=[
            pltpu.VMEM((n, d), jnp.float32),
            pltpu.SemaphoreType.DMA,
        ],
    )(X, W, b.reshape(1, d))
    return out


# speedup vs baseline: 1.3378x; 1.3378x over previous
import jax
import jax.numpy as jnp
from jax.experimental import pallas as pl
from jax.experimental.pallas import tpu as pltpu

_CHUNK = 8


def _dma_kernel(x_ref, w_ref, b_ref, o_ref, y_ref, sem):
    y = jax.lax.dot_general(
        x_ref[...], w_ref[...], (((1,), (1,)), ((), ())),
        preferred_element_type=jnp.float32)
    y_ref[...] = jnp.maximum(y + b_ref[...], 0.0)
    n = x_ref.shape[0]

    def issue(i):
        return pltpu.make_async_copy(y_ref, o_ref.at[i], sem)

    for j in range(_CHUNK):
        issue(j).start()

    def body(i, _):
        issue(i + _CHUNK).start()
        issue(i).wait()
        return 0

    jax.lax.fori_loop(0, n - _CHUNK, body, 0)
    for j in range(_CHUNK):
        issue(n - _CHUNK + j).wait()


def kernel(X, sample_domain, W, b):
    n, d = X.shape
    out = pl.pallas_call(
        _dma_kernel,
        in_specs=[
            pl.BlockSpec(memory_space=pltpu.VMEM),
            pl.BlockSpec(memory_space=pltpu.VMEM),
            pl.BlockSpec(memory_space=pltpu.VMEM),
        ],
        out_specs=pl.BlockSpec(memory_space=pl.ANY),
        out_shape=jax.ShapeDtypeStruct((n, n, d), jnp.float32),
        scratch_shapes=[
            pltpu.VMEM((n, d), jnp.float32),
            pltpu.SemaphoreType.DMA,
        ],
    )(X, W, b.reshape(1, d))
    return out
